# baseline (speedup 1.0000x reference)
import jax
import jax.numpy as jnp
from jax import lax
from jax.experimental import pallas as pl
from jax.experimental.pallas import tpu as pltpu
from jax.experimental.pallas import tpu_sc as plsc

_NC = 2
_NS = 16
_NW = _NC * _NS

_BUF_ROWS = 512

_U_MAIN = 3072
_U_CHUNK = 512
_I_MAIN = 31232
_I_CHUNK = 488


def _chunks(total, size):
    out = []
    off = 0
    while off < total:
        n = min(size, total - off)
        out.append((off, n))
        off += n
    return out


def _stream_copy(transfers, bufs, sems_in, sems_out):
    outs = []
    for g, (src, dst, rows) in enumerate(transfers):
        b = g % 2
        if g >= 2:
            outs[g - 2].wait()
        buf = bufs[b].at[pl.ds(0, rows)]
        cin = pltpu.make_async_copy(src, buf, sems_in[b])
        cin.start()
        cin.wait()
        cout = pltpu.make_async_copy(buf, dst, sems_out[b])
        cout.start()
        outs.append(cout)
    for c in outs[-2:]:
        c.wait()


def _main_transfers(src, dst, base, main_rows, chunk):
    return [
        (src.at[pl.ds(base + off, n)], dst.at[pl.ds(base + off, n)], n)
        for off, n in _chunks(main_rows, chunk)
    ]


def _body(u_in, i_in, u_out, i_out, buf0, buf1, si0, si1, so0, so1):
    wid = lax.axis_index("s") * _NC + lax.axis_index("c")
    bufs = (buf0, buf1)
    sems_in = (si0, si1)
    sems_out = (so0, so1)
    u_base = pl.multiple_of(wid * _U_MAIN, 8)
    i_base = pl.multiple_of(wid * _I_MAIN, 8)
    _stream_copy(
        _main_transfers(u_in, u_out, u_base, _U_MAIN, _U_CHUNK)
        + _main_transfers(i_in, i_out, i_base, _I_MAIN, _I_CHUNK),
        bufs, sems_in, sems_out,
    )

    u_tail_base = _U_MAIN * _NW
    i_tail_base = _I_MAIN * _NW

    @pl.when(wid == 0)
    def _():
        _stream_copy(
            _main_transfers(u_in, u_out, u_tail_base,
                            u_in.shape[0] - u_tail_base, _BUF_ROWS),
            bufs, sems_in, sems_out,
        )

    @pl.when(wid == 1)
    def _():
        _stream_copy(
            _main_transfers(i_in, i_out, i_tail_base,
                            i_in.shape[0] - i_tail_base, _BUF_ROWS),
            bufs, sems_in, sems_out,
        )


def kernel(user_weight, item_weight, edge_index):
    mesh = plsc.VectorSubcoreMesh(core_axis_name="c", subcore_axis_name="s")
    run = pl.kernel(
        _body,
        out_type=(
            jax.ShapeDtypeStruct(user_weight.shape, user_weight.dtype),
            jax.ShapeDtypeStruct(item_weight.shape, item_weight.dtype),
        ),
        mesh=mesh,
        scratch_types=[
            pltpu.VMEM((_BUF_ROWS, 32), jnp.float32),
            pltpu.VMEM((_BUF_ROWS, 32), jnp.float32),
            pltpu.SemaphoreType.DMA,
            pltpu.SemaphoreType.DMA,
            pltpu.SemaphoreType.DMA,
            pltpu.SemaphoreType.DMA,
        ],
    )
    return run(user_weight, item_weight)


# speedup vs baseline: 1.0666x; 1.0666x over previous
import jax
import jax.numpy as jnp
from jax.experimental import pallas as pl
from jax.experimental.pallas import tpu as pltpu

_NBUF = 6
_CHUNK = 12800


def _chunk_list(rows):
    out = []
    off = 0
    while off < rows:
        n = min(_CHUNK, rows - off)
        out.append((off, n))
        off += n
    return out


def _ring_copy(transfers, bufs, sems_in, sems_out):
    n = len(transfers)
    ins = [None] * n
    outs = [None] * n

    def start_in(g):
        src, _, rows = transfers[g]
        b = g % _NBUF
        ins[g] = pltpu.make_async_copy(src, bufs[b].at[pl.ds(0, rows)],
                                       sems_in[b])
        ins[g].start()

    for g in range(min(_NBUF, n)):
        start_in(g)
    for g in range(n):
        b = g % _NBUF
        _, dst, rows = transfers[g]
        ins[g].wait()
        outs[g] = pltpu.make_async_copy(bufs[b].at[pl.ds(0, rows)], dst,
                                        sems_out[b])
        outs[g].start()
        nxt = g + _NBUF
        if nxt < n:
            outs[g].wait()
            start_in(nxt)
    for g in range(max(0, n - _NBUF), n):
        outs[g].wait()


def _copy_body(u_in, i_in, u_out, i_out, *scratch):
    bufs = scratch[:_NBUF]
    sems_in = [scratch[_NBUF].at[k] for k in range(_NBUF)]
    sems_out = [scratch[_NBUF + 1].at[k] for k in range(_NBUF)]
    transfers = [
        (i_in.at[pl.ds(off, n)], i_out.at[pl.ds(off, n)], n)
        for off, n in _chunk_list(i_in.shape[0])
    ] + [
        (u_in.at[pl.ds(off, n)], u_out.at[pl.ds(off, n)], n)
        for off, n in _chunk_list(u_in.shape[0])
    ]
    _ring_copy(transfers, bufs, sems_in, sems_out)


def kernel(user_weight, item_weight, edge_index):
    out_shape = (
        jax.ShapeDtypeStruct(user_weight.shape, user_weight.dtype),
        jax.ShapeDtypeStruct(item_weight.shape, item_weight.dtype),
    )
    return pl.pallas_call(
        _copy_body,
        in_specs=[
            pl.BlockSpec(memory_space=pl.ANY),
            pl.BlockSpec(memory_space=pl.ANY),
        ],
        out_specs=[
            pl.BlockSpec(memory_space=pl.ANY),
            pl.BlockSpec(memory_space=pl.ANY),
        ],
        out_shape=out_shape,
        scratch_shapes=[pltpu.VMEM((_CHUNK, 32), jnp.float32)] * _NBUF
        + [pltpu.SemaphoreType.DMA((_NBUF,)),
           pltpu.SemaphoreType.DMA((_NBUF,))],
    )(user_weight, item_weight)
